# baseline (speedup 1.0000x reference)
import functools

import jax
import jax.numpy as jnp
from jax import lax
from jax.experimental import pallas as pl
from jax.experimental.pallas import tpu as pltpu
from jax.experimental.pallas import tpu_sc as plsc

L = 16
NC = 2
NS = 16
NW = NC * NS


def _sc_lookup(n_half: int, vocab: int, half: int):
    per_w = n_half // NW
    mesh = plsc.VectorSubcoreMesh(core_axis_name="c", subcore_axis_name="s")

    @functools.partial(
        pl.kernel,
        out_type=jax.ShapeDtypeStruct((n_half,), jnp.int32),
        mesh=mesh,
        scratch_types=[
            pltpu.VMEM((vocab,), jnp.int32),
            pltpu.VMEM((per_w,), jnp.int32),
            pltpu.VMEM((per_w,), jnp.int32),
        ],
        compiler_params=pltpu.CompilerParams(needs_layout_passes=False),
    )
    def sc_kernel(ids_hbm, table_hbm, next_hbm, table_v, ids_v, next_v):
        wid = lax.axis_index("s") * NC + lax.axis_index("c")
        base = wid * per_w
        pltpu.sync_copy(table_hbm, table_v)
        pltpu.sync_copy(ids_hbm.at[pl.ds(half * n_half + base, per_w)], ids_v)

        @pl.loop(0, per_w // L, unroll=8)
        def _(j):
            ids16 = ids_v[pl.ds(j * L, L)]
            next_v[pl.ds(j * L, L)] = plsc.load_gather(table_v, [ids16])

        pltpu.sync_copy(next_v, next_hbm.at[pl.ds(base, per_w)])

    return sc_kernel


def _tc_onehot_half(batch: int, seq: int, vocab: int, half: int):
    hb = batch // 2
    row0 = half * hb

    RB = 8

    def body(fill_ref, nx_ref, *refs):
        out_ref = refs[-1]
        f0 = fill_ref[0]
        f1 = fill_ref[1]
        vio = lax.broadcasted_iota(jnp.int32, (vocab, 128), 0)
        for rr in range(RB):
            for g in range(seq // 128):
                row = nx_ref[rr, pl.ds(g, 1), :]
                bc = jnp.broadcast_to(row, (vocab, 128))
                out_ref[rr, :, pl.ds(g * 128, 128)] = jnp.where(
                    bc == vio, f1, f0)

    in_specs = [
        pl.BlockSpec(memory_space=pltpu.SMEM),
        pl.BlockSpec((RB, seq // 128, 128), lambda b: (b, 0, 0)),
    ]
    kwargs = {}
    if half:
        in_specs.append(pl.BlockSpec(memory_space=pltpu.MemorySpace.HBM))
        kwargs["input_output_aliases"] = {2: 0}
    return pl.pallas_call(
        body,
        grid=(hb // RB,),
        in_specs=in_specs,
        out_specs=pl.BlockSpec((RB, vocab, seq),
                               lambda b: (b + row0 // RB, 0, 0)),
        out_shape=jax.ShapeDtypeStruct((batch, vocab, seq), jnp.float32),
        **kwargs,
    )


def kernel(input_ids, transition_table, fill_values):
    batch, seq = input_ids.shape
    vocab = transition_table.shape[0]
    n = batch * seq
    n_half = n // 2
    ids_flat = input_ids.reshape(n)
    next0 = _sc_lookup(n_half, vocab, 0)(ids_flat, transition_table)
    next1 = _sc_lookup(n_half, vocab, 1)(ids_flat, transition_table)
    nxa = next0.reshape(batch // 2, seq // 128, 128)
    nxb = next1.reshape(batch // 2, seq // 128, 128)
    out_t = _tc_onehot_half(batch, seq, vocab, 0)(fill_values, nxa)
    out_t = _tc_onehot_half(batch, seq, vocab, 1)(fill_values, nxb, out_t)
    return jnp.transpose(out_t, (0, 2, 1))


# speedup vs baseline: 1.0152x; 1.0152x over previous
import functools

import jax
import jax.numpy as jnp
from jax import lax
from jax.experimental import pallas as pl
from jax.experimental.pallas import tpu as pltpu
from jax.experimental.pallas import tpu_sc as plsc

L = 16
NC = 2
NS = 16
NW = NC * NS


def _sc_lookup(n_half: int, vocab: int, half: int):
    per_w = n_half // NW
    mesh = plsc.VectorSubcoreMesh(core_axis_name="c", subcore_axis_name="s")

    @functools.partial(
        pl.kernel,
        out_type=jax.ShapeDtypeStruct((n_half,), jnp.int32),
        mesh=mesh,
        scratch_types=[
            pltpu.VMEM((vocab,), jnp.int32),
            pltpu.VMEM((per_w,), jnp.int32),
            pltpu.VMEM((per_w,), jnp.int32),
        ],
        compiler_params=pltpu.CompilerParams(needs_layout_passes=False),
    )
    def sc_kernel(ids_hbm, table_hbm, next_hbm, table_v, ids_v, next_v):
        wid = lax.axis_index("s") * NC + lax.axis_index("c")
        base = wid * per_w
        pltpu.sync_copy(table_hbm, table_v)
        pltpu.sync_copy(ids_hbm.at[pl.ds(half * n_half + base, per_w)], ids_v)

        @pl.loop(0, per_w // L, unroll=8)
        def _(j):
            ids16 = ids_v[pl.ds(j * L, L)]
            next_v[pl.ds(j * L, L)] = plsc.load_gather(table_v, [ids16])

        pltpu.sync_copy(next_v, next_hbm.at[pl.ds(base, per_w)])

    return sc_kernel


def _tc_onehot_half(batch: int, seq: int, vocab: int, half: int):
    hb = batch // 2
    row0 = half * hb

    RB = 4

    def body(fill_ref, nx_ref, *refs):
        out_ref = refs[-1]
        f0 = fill_ref[0]
        f1 = fill_ref[1]
        vio = lax.broadcasted_iota(jnp.int32, (vocab, 128), 0)
        for rr in range(RB):
            for g in range(seq // 128):
                row = nx_ref[rr, pl.ds(g, 1), :]
                bc = jnp.broadcast_to(row, (vocab, 128))
                out_ref[rr, :, pl.ds(g * 128, 128)] = jnp.where(
                    bc == vio, f1, f0)

    in_specs = [
        pl.BlockSpec(memory_space=pltpu.SMEM),
        pl.BlockSpec((RB, seq // 128, 128), lambda b: (b, 0, 0)),
    ]
    kwargs = {}
    if half:
        in_specs.append(pl.BlockSpec(memory_space=pltpu.MemorySpace.HBM))
        kwargs["input_output_aliases"] = {2: 0}
    return pl.pallas_call(
        body,
        grid=(hb // RB,),
        in_specs=in_specs,
        out_specs=pl.BlockSpec((RB, vocab, seq),
                               lambda b: (b + row0 // RB, 0, 0)),
        out_shape=jax.ShapeDtypeStruct((batch, vocab, seq), jnp.float32),
        **kwargs,
    )


def kernel(input_ids, transition_table, fill_values):
    batch, seq = input_ids.shape
    vocab = transition_table.shape[0]
    n = batch * seq
    n_half = n // 2
    ids_flat = input_ids.reshape(n)
    next0 = _sc_lookup(n_half, vocab, 0)(ids_flat, transition_table)
    next1 = _sc_lookup(n_half, vocab, 1)(ids_flat, transition_table)
    nxa = next0.reshape(batch // 2, seq // 128, 128)
    nxb = next1.reshape(batch // 2, seq // 128, 128)
    out_t = _tc_onehot_half(batch, seq, vocab, 0)(fill_values, nxa)
    out_t = _tc_onehot_half(batch, seq, vocab, 1)(fill_values, nxb, out_t)
    return jnp.transpose(out_t, (0, 2, 1))


# speedup vs baseline: 1.1220x; 1.1052x over previous
import functools

import jax
import jax.numpy as jnp
from jax import lax
from jax.experimental import pallas as pl
from jax.experimental.pallas import tpu as pltpu
from jax.experimental.pallas import tpu_sc as plsc

L = 16
NC = 2
NS = 16
NW = NC * NS


def _sc_lookup(n_half: int, vocab: int, half: int):
    per_w = n_half // NW
    mesh = plsc.VectorSubcoreMesh(core_axis_name="c", subcore_axis_name="s")

    @functools.partial(
        pl.kernel,
        out_type=jax.ShapeDtypeStruct((n_half,), jnp.int32),
        mesh=mesh,
        scratch_types=[
            pltpu.VMEM((vocab,), jnp.int32),
            pltpu.VMEM((per_w,), jnp.int32),
            pltpu.VMEM((per_w,), jnp.int32),
        ],
        compiler_params=pltpu.CompilerParams(needs_layout_passes=False),
    )
    def sc_kernel(ids_hbm, table_hbm, next_hbm, table_v, ids_v, next_v):
        wid = lax.axis_index("s") * NC + lax.axis_index("c")
        base = wid * per_w
        pltpu.sync_copy(table_hbm, table_v)
        pltpu.sync_copy(ids_hbm.at[pl.ds(half * n_half + base, per_w)], ids_v)

        @pl.loop(0, per_w // L, unroll=8)
        def _(j):
            ids16 = ids_v[pl.ds(j * L, L)]
            next_v[pl.ds(j * L, L)] = plsc.load_gather(table_v, [ids16])

        pltpu.sync_copy(next_v, next_hbm.at[pl.ds(base, per_w)])

    return sc_kernel


def _tc_onehot_full(batch: int, seq: int, vocab: int):
    RB = 4

    def body(fill_ref, nx_ref, out_ref):
        f0 = fill_ref[0]
        f1 = fill_ref[1]
        vio = lax.broadcasted_iota(jnp.int32, (vocab, 128), 0)
        for rr in range(RB):
            for g in range(seq // 128):
                row = nx_ref[rr, pl.ds(g, 1), :]
                bc = jnp.broadcast_to(row, (vocab, 128))
                out_ref[rr, :, pl.ds(g * 128, 128)] = jnp.where(
                    bc == vio, f1, f0)

    return pl.pallas_call(
        body,
        grid=(batch // RB,),
        in_specs=[
            pl.BlockSpec(memory_space=pltpu.SMEM),
            pl.BlockSpec((RB, seq // 128, 128), lambda b: (b, 0, 0)),
        ],
        out_specs=pl.BlockSpec((RB, vocab, seq), lambda b: (b, 0, 0)),
        out_shape=jax.ShapeDtypeStruct((batch, vocab, seq), jnp.float32),
    )


def _tc_onehot_half(batch: int, seq: int, vocab: int, half: int):
    hb = batch // 2
    row0 = half * hb

    RB = 4

    def body(fill_ref, nx_ref, *refs):
        out_ref = refs[-1]
        f0 = fill_ref[0]
        f1 = fill_ref[1]
        vio = lax.broadcasted_iota(jnp.int32, (vocab, 128), 0)
        for rr in range(RB):
            for g in range(seq // 128):
                row = nx_ref[rr, pl.ds(g, 1), :]
                bc = jnp.broadcast_to(row, (vocab, 128))
                out_ref[rr, :, pl.ds(g * 128, 128)] = jnp.where(
                    bc == vio, f1, f0)

    in_specs = [
        pl.BlockSpec(memory_space=pltpu.SMEM),
        pl.BlockSpec((RB, seq // 128, 128), lambda b: (b, 0, 0)),
    ]
    kwargs = {}
    if half:
        in_specs.append(pl.BlockSpec(memory_space=pltpu.MemorySpace.HBM))
        kwargs["input_output_aliases"] = {2: 0}
    return pl.pallas_call(
        body,
        grid=(hb // RB,),
        in_specs=in_specs,
        out_specs=pl.BlockSpec((RB, vocab, seq),
                               lambda b: (b + row0 // RB, 0, 0)),
        out_shape=jax.ShapeDtypeStruct((batch, vocab, seq), jnp.float32),
        **kwargs,
    )


def kernel(input_ids, transition_table, fill_values):
    batch, seq = input_ids.shape
    vocab = transition_table.shape[0]
    n = batch * seq
    n_half = n // 2
    ids_flat = input_ids.reshape(n)
    next_flat = _sc_lookup(n, vocab, 0)(ids_flat, transition_table)
    nx = next_flat.reshape(batch, seq // 128, 128)
    out_t = _tc_onehot_full(batch, seq, vocab)(fill_values, nx)
    return jnp.transpose(out_t, (0, 2, 1))
